# baseline (speedup 1.0000x reference)
import functools

import jax
import jax.numpy as jnp
from jax import lax
from jax.experimental import pallas as pl
from jax.experimental.pallas import tpu as pltpu
from jax.experimental.pallas import tpu_sc as plsc


def _gather_kernel(nw, b_dim, h_dim, d, bblk):
    mesh = plsc.VectorSubcoreMesh(core_axis_name="c", subcore_axis_name="s")
    nbt = b_dim // bblk

    @functools.partial(
        pl.kernel,
        mesh=mesh,
        out_type=jax.ShapeDtypeStruct((h_dim, d // 8, nbt, 8, bblk), jnp.float32),
        compiler_params=pltpu.CompilerParams(
            use_tc_tiling_on_sc=False, needs_layout_passes=False),
        scratch_types=[
            pltpu.VMEM((h_dim, bblk), jnp.int32),
            *[pltpu.VMEM((bblk, d), jnp.float32) for _ in range(5)],
            *[pltpu.VMEM((d // 8, 8, bblk + 1), jnp.float32) for _ in range(5)],
            *[pltpu.SemaphoreType.DMA for _ in range(10)],
        ],
    )
    def k(x_hbm, table_hbm, out_hbm, idx_v, *bufs_and_sems):
        gbuf = bufs_and_sems[0:5]
        tbuf = bufs_and_sems[5:10]
        gsem = bufs_and_sems[10:15]
        wsem = bufs_and_sems[15:20]
        wid = lax.axis_index("s") * 2 + lax.axis_index("c")
        pltpu.sync_copy(x_hbm.at[wid], idx_v)

        def start_gather(h, b):
            pltpu.async_copy(table_hbm.at[idx_v.at[h]], gbuf[b], gsem[b])

        def wait_gather(h, b):
            pltpu.make_async_copy(
                table_hbm.at[idx_v.at[h]], gbuf[b], gsem[b]).wait()

        def transpose(b):
            @plsc.parallel_loop(0, bblk, unroll=2)
            def _t(r):
                lanes = lax.iota(jnp.int32, 16)
                half = lanes >> 3
                i1 = lanes & 7
                i2 = jnp.full((16,), r, jnp.int32)
                for k in range(d // 16):
                    vals = gbuf[b][r, pl.ds(16 * k, 16)]
                    plsc.store_scatter(
                        tbuf[b], [half + (2 * k), i1, i2], vals)

        def start_write(h, b):
            pltpu.async_copy(
                tbuf[b].at[:, :, pl.ds(0, bblk)], out_hbm.at[h, :, wid],
                wsem[b])

        def wait_write(h, b):
            pltpu.make_async_copy(
                tbuf[b].at[:, :, pl.ds(0, bblk)], out_hbm.at[h, :, wid],
                wsem[b]).wait()

        nbuf = 5
        for b in range(nbuf):
            start_gather(b, b)
        for b in range(nbuf):
            wait_gather(b, b)
            transpose(b)
            start_write(b, b)
            start_gather(b + nbuf, b)

        def body(g, carry):
            for b in range(nbuf):
                h = nbuf * g + b
                wait_gather(h, b)
                wait_write(h - nbuf, b)
                transpose(b)
                start_write(h, b)
                start_gather(h + nbuf, b)
            return carry

        lax.fori_loop(1, h_dim // nbuf - 1, body, 0)

        last = h_dim - nbuf
        for b in range(nbuf):
            h = last + b
            wait_gather(h, b)
            wait_write(h - nbuf, b)
            transpose(b)
            start_write(h, b)
        for b in range(nbuf):
            wait_write(last + b, b)

    return k


def kernel(x, table):
    b, h = x.shape
    v, d = table.shape
    nw = 32
    bblk = b // nw
    xr = x.reshape(nw, bblk, h).transpose(0, 2, 1).astype(jnp.int32)
    out5 = _gather_kernel(nw, b, h, d, bblk)(xr, table)
    return out5.transpose(2, 4, 0, 1, 3).reshape(b, h, d)


# speedup vs baseline: 1.0034x; 1.0034x over previous
import functools

import jax
import jax.numpy as jnp
from jax import lax
from jax.experimental import pallas as pl
from jax.experimental.pallas import tpu as pltpu
from jax.experimental.pallas import tpu_sc as plsc


def _gather_kernel(nw, b_dim, h_dim, d, bblk):
    mesh = plsc.VectorSubcoreMesh(core_axis_name="c", subcore_axis_name="s")
    nbt = b_dim // bblk

    @functools.partial(
        pl.kernel,
        mesh=mesh,
        out_type=jax.ShapeDtypeStruct((h_dim, d // 8, nbt, 8, bblk), jnp.float32),
        compiler_params=pltpu.CompilerParams(
            use_tc_tiling_on_sc=False, needs_layout_passes=False),
        scratch_types=[
            pltpu.VMEM((h_dim, bblk), jnp.int32),
            *[pltpu.VMEM((bblk, d), jnp.float32) for _ in range(4)],
            *[pltpu.VMEM((d // 8, 8, bblk + 1), jnp.float32) for _ in range(4)],
            *[pltpu.SemaphoreType.DMA for _ in range(8)],
        ],
    )
    def k(x_hbm, table_hbm, out_hbm, idx_v, *bufs_and_sems):
        gbuf = bufs_and_sems[0:4]
        tbuf = bufs_and_sems[4:8]
        gsem = bufs_and_sems[8:12]
        wsem = bufs_and_sems[12:16]
        wid = lax.axis_index("s") * 2 + lax.axis_index("c")
        pltpu.sync_copy(x_hbm.at[wid], idx_v)

        def start_gather(h, b):
            pltpu.async_copy(table_hbm.at[idx_v.at[h]], gbuf[b], gsem[b])

        def wait_gather(h, b):
            pltpu.make_async_copy(
                table_hbm.at[idx_v.at[h]], gbuf[b], gsem[b]).wait()

        def transpose(b):
            @plsc.parallel_loop(0, bblk, unroll=2)
            def _t(r):
                lanes = lax.iota(jnp.int32, 16)
                half = lanes >> 3
                i1 = lanes & 7
                i2 = jnp.full((16,), r, jnp.int32)
                for k in range(d // 16):
                    vals = gbuf[b][r, pl.ds(16 * k, 16)]
                    plsc.store_scatter(
                        tbuf[b], [half + (2 * k), i1, i2], vals)

        def start_write(h, b):
            pltpu.async_copy(
                tbuf[b].at[:, :, pl.ds(0, bblk)], out_hbm.at[h, :, wid],
                wsem[b])

        def wait_write(h, b):
            pltpu.make_async_copy(
                tbuf[b].at[:, :, pl.ds(0, bblk)], out_hbm.at[h, :, wid],
                wsem[b]).wait()

        nbuf = 4
        for b in range(nbuf):
            start_gather(b, b)
        for b in range(nbuf):
            wait_gather(b, b)
            transpose(b)
            start_write(b, b)
            start_gather(b + nbuf, b)

        def body(g, carry):
            for b in range(nbuf):
                h = nbuf * g + b
                wait_gather(h, b)
                wait_write(h - nbuf, b)
                transpose(b)
                start_write(h, b)
                start_gather(h + nbuf, b)
            return carry

        lax.fori_loop(1, h_dim // nbuf - 1, body, 0)

        last = h_dim - nbuf
        for b in range(nbuf):
            h = last + b
            wait_gather(h, b)
            wait_write(h - nbuf, b)
            transpose(b)
            start_write(h, b)
        for b in range(nbuf):
            wait_write(last + b, b)

    return k


def kernel(x, table):
    b, h = x.shape
    v, d = table.shape
    nw = 32
    bblk = b // nw
    xr = x.reshape(nw, bblk, h).transpose(0, 2, 1).astype(jnp.int32)
    out5 = _gather_kernel(nw, b, h, d, bblk)(xr, table)
    return out5.transpose(2, 4, 0, 1, 3).reshape(b, h, d)
